# baseline (speedup 1.0000x reference)
import functools

import jax
import jax.numpy as jnp
from jax import lax
from jax.experimental import pallas as pl
from jax.experimental.pallas import tpu as pltpu
from jax.experimental.pallas import tpu_sc as plsc

N = 10000
E = 320000
D = 128
P = 5000

NC = 2
NS = 16
NW = NC * NS
CHUNK = 128
NBUF = 2
EPW = -(-E // NW)
NCHUNK = NBUF * (-(-EPW // (CHUNK * NBUF)))
EPW_PAD = NCHUNK * CHUNK
E_PAD = EPW_PAD * NW
HALF = 40
TRASH = 10240 + 128 - 16
NROWS = 10240
RPT = NROWS // NS
HI = NROWS // 128
BE = 3200
NSTEP = E // BE

_mesh = plsc.VectorSubcoreMesh(core_axis_name="c", subcore_axis_name="s")


@functools.partial(
    pl.kernel,
    out_type=[
        jax.ShapeDtypeStruct((NROWS, D), jnp.float32),
        jax.ShapeDtypeStruct((NROWS, D), jnp.float32),
    ],
    mesh=_mesh,
    scratch_types=[
        pltpu.VMEM_SHARED((NROWS, D), jnp.float32),
        pltpu.VMEM((EPW_PAD,), jnp.int32),
        pltpu.VMEM((160,), jnp.int32),
        pltpu.VMEM((NCHUNK, CHUNK), jnp.int32),
        pltpu.VMEM((NCHUNK, CHUNK), jnp.int32),
        pltpu.VMEM((CHUNK, D), jnp.float32),
        pltpu.SemaphoreType.DMA,
    ],
)
def _edge_kernel(pk_hbm, h2_hbm, zeros_hbm, out0, out1,
                 acc_sh, pk_v, stag, src_c, dst_c, buf, sem):
    cid = lax.axis_index("c")
    sid = lax.axis_index("s")
    wid = cid * NS + sid
    pltpu.sync_copy(zeros_hbm.at[pl.ds(sid * RPT, RPT)],
                    acc_sh.at[pl.ds(sid * RPT, RPT)])
    pltpu.sync_copy(pk_hbm.at[wid], pk_v)

    iota = lax.broadcasted_iota(jnp.int32, (16,), 0)
    fillpk = jnp.full((16,), (N << 16) | N, jnp.int32)
    deadpk = jnp.int32(P << 16)

    stages = []
    for k in (2, 4, 8, 16):
        j = k >> 1
        while j >= 1:
            bitj = (iota >> j.bit_length() - 1) & 1
            bitk = (iota >> k.bit_length() - 1) & 1
            tm = 1 - (bitj ^ bitk)
            stages.append((j, tm))
            j >>= 1

    def flush(rr):
        for k in range(CHUNK // 16):
            row = stag[pl.ds(k * 16, 16)]
            src_c[rr, pl.ds(k * 16, 16)] = row >> 16
            dst_c[rr, pl.ds(k * 16, 16)] = row & 0xFFFF
        stag[pl.ds(0, 16)] = stag[pl.ds(128, 16)]

    def compact(v, carry):
        sc, r = carry
        pk = pk_v[pl.ds(v * 16, 16)]
        m = jnp.where(pk < deadpk, 1, 0)
        sv = pk
        for j, tm in stages:
            partner = sv[jnp.clip(iota ^ j, 0, 15)]
            lo = jnp.minimum(sv, partner)
            hi = jnp.maximum(sv, partner)
            sv = hi - tm * (hi - lo)
        pref = m
        for step in (1, 2, 4, 8):
            shifted = pref[jnp.clip(iota - step, 0, 15)]
            pref = pref + jnp.where(iota >= step, shifted, 0)
        pc = pref[15]
        stag[pl.ds(sc, 16)] = sv
        sc = sc + pc

        @pl.when(sc >= CHUNK)
        def _():
            flush(r)

        wrapped = jnp.where(sc >= CHUNK, 1, 0)
        return (sc - CHUNK * wrapped, r + wrapped)

    sc, r = lax.fori_loop(0, EPW_PAD // 16, compact, (0, 0))

    @pl.when(sc > 0)
    def _():
        stag[pl.ds(sc, 16)] = fillpk
        for k in range(1, CHUNK // 16):
            @pl.when(sc <= k * 16)
            def _():
                stag[pl.ds(k * 16, 16)] = fillpk
        flush(r)

    nch = r + jnp.where(sc > 0, 1, 0)
    plsc.subcore_barrier()

    def stream_body(j, _):
        pltpu.async_copy(h2_hbm.at[src_c.at[j]], buf, sem).wait()
        pltpu.sync_copy(buf, acc_sh.at[dst_c.at[j]], add=True)
        return 0

    lax.fori_loop(0, nch, stream_body, 0)
    plsc.subcore_barrier()

    @pl.when(cid == 0)
    def _():
        pltpu.sync_copy(acc_sh.at[pl.ds(sid * RPT, RPT)],
                        out0.at[pl.ds(sid * RPT, RPT)])

    @pl.when(cid == 1)
    def _():
        pltpu.sync_copy(acc_sh.at[pl.ds(sid * RPT, RPT)],
                        out1.at[pl.ds(sid * RPT, RPT)])


def _hist_body(dst_ref, c_ref):
    step = pl.program_id(0)
    d = dst_ref[0, 0, :]
    hi = (d >> 7)[:, None]
    lo = (d & 127)[:, None]
    oh = (hi == lax.broadcasted_iota(jnp.int32, (BE, HI), 1)).astype(jnp.bfloat16)
    ol = (lo == lax.broadcasted_iota(jnp.int32, (BE, 128), 1)).astype(jnp.bfloat16)
    c = lax.dot_general(oh, ol, (((0,), (0,)), ((), ())),
                        preferred_element_type=jnp.float32)

    @pl.when(step == 0)
    def _():
        c_ref[...] = c

    @pl.when(step != 0)
    def _():
        c_ref[...] += c


def _prep_body(x_ref, w_ref, c_ref, h2_ref):
    deg = c_ref[0].reshape(_BM) + 1.0
    dinv = lax.rsqrt(deg)
    h = jnp.dot(x_ref[...], w_ref[...], preferred_element_type=jnp.float32)
    h2_ref[...] = h * dinv[:, None]


def _final_body(h2_ref, a0_ref, a1_ref, c_ref, b_ref, o_ref):
    deg = c_ref[0].reshape(_BM) + 1.0
    dinv = lax.rsqrt(deg)
    s = a0_ref[...] + a1_ref[...] + h2_ref[...]
    o_ref[...] = jnp.maximum(s * dinv[:, None] + b_ref[...], 0.0)


_BM = 512
_BMC = _BM // 128


def kernel(x_skip, e_skip, indices, x, W, b):
    del x_skip, indices
    src = e_skip[0].astype(jnp.int32)
    dst = e_skip[1].astype(jnp.int32)
    pk = (src << 16) | dst
    pad = jnp.full((E_PAD - E,), (N << 16) | N, jnp.int32)
    pkp = jnp.concatenate([pk, pad]).reshape(NW, EPW_PAD)
    zerosD = jnp.zeros((NROWS, D), jnp.float32)
    x_pad = jnp.concatenate(
        [x.astype(jnp.float32), jnp.zeros((NROWS - P, D), jnp.float32)])

    c_deg = pl.pallas_call(
        _hist_body,
        grid=(NSTEP,),
        in_specs=[pl.BlockSpec((1, 1, BE), lambda i: (i, 0, 0))],
        out_specs=pl.BlockSpec((HI, 128), lambda i: (0, 0)),
        out_shape=jax.ShapeDtypeStruct((HI, 128), jnp.float32),
    )(dst.reshape(NSTEP, 1, BE))
    c3 = c_deg.reshape(NROWS // _BM, _BMC, 128)

    h2 = pl.pallas_call(
        _prep_body,
        grid=(NROWS // _BM,),
        in_specs=[
            pl.BlockSpec((_BM, D), lambda i: (i, 0)),
            pl.BlockSpec((D, D), lambda i: (0, 0)),
            pl.BlockSpec((1, _BMC, 128), lambda i: (i, 0, 0)),
        ],
        out_specs=pl.BlockSpec((_BM, D), lambda i: (i, 0)),
        out_shape=jax.ShapeDtypeStruct((NROWS, D), jnp.float32),
    )(x_pad, W.astype(jnp.float32), c3)

    acc0, acc1 = _edge_kernel(pkp, h2, zerosD)

    out_pad = pl.pallas_call(
        _final_body,
        grid=(NROWS // _BM,),
        in_specs=[
            pl.BlockSpec((_BM, D), lambda i: (i, 0)),
            pl.BlockSpec((_BM, D), lambda i: (i, 0)),
            pl.BlockSpec((_BM, D), lambda i: (i, 0)),
            pl.BlockSpec((1, _BMC, 128), lambda i: (i, 0, 0)),
            pl.BlockSpec((1, D), lambda i: (0, 0)),
        ],
        out_specs=pl.BlockSpec((_BM, D), lambda i: (i, 0)),
        out_shape=jax.ShapeDtypeStruct((NROWS, D), jnp.float32),
    )(h2, acc0, acc1, c3, b.reshape(1, D).astype(jnp.float32))

    return (out_pad[:N], e_skip)


# speedup vs baseline: 1.0908x; 1.0908x over previous
import functools

import jax
import jax.numpy as jnp
from jax import lax
from jax.experimental import pallas as pl
from jax.experimental.pallas import tpu as pltpu
from jax.experimental.pallas import tpu_sc as plsc

N = 10000
E = 320000
D = 128
P = 5000

NC = 2
NS = 16
NW = NC * NS
CHUNK = 128
EPW = -(-E // NW)
NCHUNK = -(-EPW // CHUNK)
EPW_PAD = NCHUNK * CHUNK
E_PAD = EPW_PAD * NW
HALF = 40
TRASH = 10240 + 128 - 16
NROWS = 10240
RPT = NROWS // NS
HI = NROWS // 128
BE = 3200
NSTEP = E // BE

_mesh = plsc.VectorSubcoreMesh(core_axis_name="c", subcore_axis_name="s")


@functools.partial(
    pl.kernel,
    out_type=[
        jax.ShapeDtypeStruct((NROWS, D), jnp.float32),
        jax.ShapeDtypeStruct((NROWS, D), jnp.float32),
    ],
    mesh=_mesh,
    scratch_types=[
        pltpu.VMEM_SHARED((NROWS, D), jnp.float32),
        pltpu.VMEM((NCHUNK, CHUNK), jnp.int32),
        pltpu.VMEM((NCHUNK, CHUNK), jnp.int32),
        pltpu.VMEM((CHUNK, D), jnp.float32),
        pltpu.SemaphoreType.DMA,
    ],
)
def _edge_kernel(src_hbm, dst_hbm, h2_hbm, zeros_hbm, out0, out1,
                 acc_sh, src_v, dst_v, rows_v, sem):
    cid = lax.axis_index("c")
    sid = lax.axis_index("s")
    wid = cid * NS + sid
    pltpu.sync_copy(zeros_hbm.at[pl.ds(sid * RPT, RPT)],
                    acc_sh.at[pl.ds(sid * RPT, RPT)])
    pltpu.sync_copy(src_hbm.at[wid], src_v)
    pltpu.sync_copy(dst_hbm.at[wid], dst_v)
    plsc.subcore_barrier()

    def body(j):
        pltpu.async_copy(h2_hbm.at[src_v.at[j]], rows_v, sem).wait()
        pltpu.sync_copy(rows_v, acc_sh.at[dst_v.at[j]], add=True)

    pl.loop(0, NCHUNK)(body)
    plsc.subcore_barrier()

    @pl.when(cid == 0)
    def _():
        pltpu.sync_copy(acc_sh.at[pl.ds(sid * RPT, RPT)],
                        out0.at[pl.ds(sid * RPT, RPT)])

    @pl.when(cid == 1)
    def _():
        pltpu.sync_copy(acc_sh.at[pl.ds(sid * RPT, RPT)],
                        out1.at[pl.ds(sid * RPT, RPT)])


def _hist_body(dst_ref, c_ref):
    step = pl.program_id(0)
    d = dst_ref[0, 0, :]
    hi = (d >> 7)[:, None]
    lo = (d & 127)[:, None]
    oh = (hi == lax.broadcasted_iota(jnp.int32, (BE, HI), 1)).astype(jnp.bfloat16)
    ol = (lo == lax.broadcasted_iota(jnp.int32, (BE, 128), 1)).astype(jnp.bfloat16)
    c = lax.dot_general(oh, ol, (((0,), (0,)), ((), ())),
                        preferred_element_type=jnp.float32)

    @pl.when(step == 0)
    def _():
        c_ref[...] = c

    @pl.when(step != 0)
    def _():
        c_ref[...] += c


def _prep_body(x_ref, w_ref, c_ref, h2_ref):
    deg = c_ref[0].reshape(_BM) + 1.0
    dinv = lax.rsqrt(deg)
    h = jnp.dot(x_ref[...], w_ref[...], preferred_element_type=jnp.float32)
    h2_ref[...] = h * dinv[:, None]


def _final_body(h2_ref, a0_ref, a1_ref, c_ref, b_ref, o_ref):
    deg = c_ref[0].reshape(_BM) + 1.0
    dinv = lax.rsqrt(deg)
    s = a0_ref[...] + a1_ref[...] + h2_ref[...]
    o_ref[...] = jnp.maximum(s * dinv[:, None] + b_ref[...], 0.0)


_BM = 512
_BMC = _BM // 128


def kernel(x_skip, e_skip, indices, x, W, b):
    del x_skip, indices
    src = e_skip[0].astype(jnp.int32)
    dst = e_skip[1].astype(jnp.int32)
    pad = N + (jnp.arange(E_PAD - E, dtype=jnp.int32) % (NROWS - N))
    srcp = jnp.concatenate([src, pad]).reshape(NW, NCHUNK, CHUNK)
    dstp = jnp.concatenate([dst, pad]).reshape(NW, NCHUNK, CHUNK)
    zerosD = jnp.zeros((NROWS, D), jnp.float32)
    x_pad = jnp.concatenate(
        [x.astype(jnp.float32), jnp.zeros((NROWS - P, D), jnp.float32)])

    c_deg = pl.pallas_call(
        _hist_body,
        grid=(NSTEP,),
        in_specs=[pl.BlockSpec((1, 1, BE), lambda i: (i, 0, 0))],
        out_specs=pl.BlockSpec((HI, 128), lambda i: (0, 0)),
        out_shape=jax.ShapeDtypeStruct((HI, 128), jnp.float32),
    )(dst.reshape(NSTEP, 1, BE))
    c3 = c_deg.reshape(NROWS // _BM, _BMC, 128)

    h2 = pl.pallas_call(
        _prep_body,
        grid=(NROWS // _BM,),
        in_specs=[
            pl.BlockSpec((_BM, D), lambda i: (i, 0)),
            pl.BlockSpec((D, D), lambda i: (0, 0)),
            pl.BlockSpec((1, _BMC, 128), lambda i: (i, 0, 0)),
        ],
        out_specs=pl.BlockSpec((_BM, D), lambda i: (i, 0)),
        out_shape=jax.ShapeDtypeStruct((NROWS, D), jnp.float32),
    )(x_pad, W.astype(jnp.float32), c3)

    acc0, acc1 = _edge_kernel(srcp, dstp, h2, zerosD)

    out_pad = pl.pallas_call(
        _final_body,
        grid=(NROWS // _BM,),
        in_specs=[
            pl.BlockSpec((_BM, D), lambda i: (i, 0)),
            pl.BlockSpec((_BM, D), lambda i: (i, 0)),
            pl.BlockSpec((_BM, D), lambda i: (i, 0)),
            pl.BlockSpec((1, _BMC, 128), lambda i: (i, 0, 0)),
            pl.BlockSpec((1, D), lambda i: (0, 0)),
        ],
        out_specs=pl.BlockSpec((_BM, D), lambda i: (i, 0)),
        out_shape=jax.ShapeDtypeStruct((NROWS, D), jnp.float32),
    )(h2, acc0, acc1, c3, b.reshape(1, D).astype(jnp.float32))

    return (out_pad[:N], e_skip)
